# baseline (speedup 1.0000x reference)
import functools

import jax
import jax.numpy as jnp
from jax import lax
from jax.experimental import pallas as pl
from jax.experimental.pallas import tpu as pltpu
from jax.experimental.pallas import tpu_sc as plsc

B, L, H = 4096, 50, 128
UEMB = 512
NBAG = 7
NCHUNK = 14

KR = 2
LP = 50
NC, NS = 2, 16
NW = NC * NS
CB = B // NW


def _load_row(src_ref, i):
    return tuple(src_ref[i, pl.ds(c * 16, 16)] for c in range(H // 16))


def _store_row(acc_ref, i, xs, first):
    for c in range(H // 16):
        sl = pl.ds(c * 16, 16)
        if first:
            acc_ref[i, sl] = xs[c]
        else:
            plsc.addupdate(acc_ref.at[i, sl], xs[c])


def _accum(acc_ref, src_ref, first=False):
    def load_rows(i):
        xs = _load_row(src_ref, i)
        for r in range(1, KR):
            ys = _load_row(src_ref, r * CB + i)
            xs = tuple(x + y for x, y in zip(xs, ys))
        return xs

    first_vals = load_rows(0)

    @plsc.parallel_loop(1, CB, carry=first_vals, unroll=4)
    def body(i, xs):
        _store_row(acc_ref, i - 1, xs, first)
        return load_rows(i)

    _store_row(acc_ref, CB - 1, body, first)


def _sc_body(sku_t, cat_t, url_t, idx_r, out, idx_v, buf_v, acc_v, cat_sh,
             sem0, sem1):
    cid = lax.axis_index("c")
    sid = lax.axis_index("s")
    wid = sid * NC + cid
    b0 = wid * CB

    @pl.when(sid == 0)
    def _load_cat():
        pltpu.sync_copy(cat_t, cat_sh)

    plsc.subcore_barrier()

    tables = (sku_t, sku_t, sku_t, cat_sh, cat_sh, cat_sh, url_t)
    nstep = LP // KR

    for t in range(NBAG):
        table = tables[t]
        pltpu.sync_copy(idx_r.at[t, wid], idx_v)

        def start(s_, buf, sem):
            pltpu.async_copy(
                table.at[idx_v.at[pl.ds(s_ * KR * CB, KR * CB)]],
                buf_v.at[buf], sem)

        def retire(buf, sem, first=False):
            pltpu.make_async_copy(table.at[idx_v.at[pl.ds(0, KR * CB)]],
                                  buf_v.at[buf], sem).wait()
            _accum(acc_v, buf_v.at[buf], first=first)

        start(0, 0, sem0)
        start(1, 1, sem1)
        retire(0, sem0, first=True)

        def pair(j, carry):
            s_even = 2 * j + 2
            start(s_even, 0, sem0)
            retire(1, sem1)
            start(s_even + 1, 1, sem1)
            retire(0, sem0)
            return carry

        lax.fori_loop(0, (nstep - 2) // 2, pair, 0)

        if nstep % 2 == 0:
            retire(1, sem1)
        else:
            start(nstep - 1, 0, sem0)
            retire(1, sem1)
            retire(0, sem0)

        pltpu.sync_copy(acc_v, out.at[t, pl.ds(b0, CB), :])


@functools.partial(jax.jit, static_argnames=())
def _sc_bag_sums(sku_table, cat_table, url_table, idx_r):
    mesh = plsc.VectorSubcoreMesh(core_axis_name="c", subcore_axis_name="s")
    return pl.kernel(
        _sc_body,
        out_type=jax.ShapeDtypeStruct((NBAG, B, H), jnp.float32),
        mesh=mesh,
        scratch_types=[
            pltpu.VMEM((LP * CB,), jnp.int32),
            pltpu.VMEM((2, KR * CB, H), jnp.float32),
            pltpu.VMEM((CB, H), jnp.float32),
            pltpu.VMEM_SHARED((1001, H), jnp.float32),
            pltpu.SemaphoreType.DMA,
            pltpu.SemaphoreType.DMA,
        ],
    )(sku_table, cat_table, url_table, idx_r)


def _mm(x, w):
    return lax.dot_general(x, w, (((1,), (1,)), ((), ())),
                           preferred_element_type=jnp.float32)


def _bn_mm(chunk, ci, gamma, beta, w_full):
    bn_s = 1.0 / jnp.sqrt(1.0 + 1e-5)
    g = gamma[pl.ds(ci * H, H)] * bn_s
    be = beta[pl.ds(ci * H, H)]
    x = chunk * g[None, :] + be[None, :]
    return _mm(x.astype(jnp.bfloat16), w_full[:, pl.ds(ci * H, H)])


def _tc_dense_body(actions, kdes, query, w_act, b_act, w_kde, b_kde,
                   w_query, b_query, gamma, beta, w_full, b_full, out):
    chunks = {
        0: _mm(actions[...], w_act[...]) + b_act[...][None, :],
        12: _mm(query[...], w_query[...]) + b_query[...][None, :],
    }
    for j in range(5):
        chunks[1 + j] = _mm(kdes[j], w_kde[j]) + b_kde[j][None, :]
    acc = jnp.zeros(out.shape, jnp.float32) + b_full[...][None, :]
    for ci, chunk in chunks.items():
        acc = acc + _bn_mm(chunk, ci, gamma, beta, w_full)
    out[...] = acc


def _tc_bag_body(bags, idxs, w_bag, b_bag, gamma, beta, w_full, part, out):
    f32 = jnp.float32
    means = []
    for t in range(NBAG):
        cnt = jnp.sum((idxs[t] != 0).astype(f32), axis=1)
        inv = 1.0 / jnp.maximum(cnt, 1.0)
        means.append(bags[t] * inv[:, None])

    acc = part[...]
    for j in range(6):
        chunk = _mm(means[j], w_bag[j]) + b_bag[j][None, :]
        acc = acc + _bn_mm(chunk, 6 + j, gamma, beta, w_full)
    acc = acc + _bn_mm(means[6], 13, gamma, beta, w_full)
    out[...] = acc


def _tc_encode(bags, idx_s, actions, kdes, query,
               w_act, b_act, w_kde, b_kde, w_query, b_query,
               w_bag, b_bag, gamma, beta, w_full, b_full):
    TB = 256
    grid = (B // TB,)
    batch2 = lambda i: (i, 0)
    batch3 = lambda i: (0, i, 0)
    whole = lambda i: (0,)
    part = pl.pallas_call(
        _tc_dense_body,
        grid=grid,
        in_specs=[
            pl.BlockSpec((TB, 5), batch2),
            pl.BlockSpec((5, TB, 11), batch3),
            pl.BlockSpec((TB, 64), batch2),
            pl.BlockSpec((H, 5), lambda i: (0, 0)),
            pl.BlockSpec((H,), whole),
            pl.BlockSpec((5, H, 11), lambda i: (0, 0, 0)),
            pl.BlockSpec((5, H), lambda i: (0, 0)),
            pl.BlockSpec((H, 64), lambda i: (0, 0)),
            pl.BlockSpec((H,), whole),
            pl.BlockSpec((NCHUNK * H,), whole),
            pl.BlockSpec((NCHUNK * H,), whole),
            pl.BlockSpec((UEMB, NCHUNK * H), lambda i: (0, 0)),
            pl.BlockSpec((UEMB,), whole),
        ],
        out_specs=pl.BlockSpec((TB, UEMB), batch2),
        out_shape=jax.ShapeDtypeStruct((B, UEMB), jnp.float32),
    )(actions, kdes, query, w_act, b_act, w_kde, b_kde, w_query, b_query,
      gamma, beta, w_full, b_full)
    return pl.pallas_call(
        _tc_bag_body,
        grid=grid,
        in_specs=[
            pl.BlockSpec((NBAG, TB, H), batch3),
            pl.BlockSpec((NBAG, TB, L), batch3),
            pl.BlockSpec((6, H, H), lambda i: (0, 0, 0)),
            pl.BlockSpec((6, H), lambda i: (0, 0)),
            pl.BlockSpec((NCHUNK * H,), whole),
            pl.BlockSpec((NCHUNK * H,), whole),
            pl.BlockSpec((UEMB, NCHUNK * H), lambda i: (0, 0)),
            pl.BlockSpec((TB, UEMB), batch2),
        ],
        out_specs=pl.BlockSpec((TB, UEMB), batch2),
        out_shape=jax.ShapeDtypeStruct((B, UEMB), jnp.float32),
    )(bags, idx_s, w_bag, b_bag, gamma, beta, w_full, part)


def kernel(actions, sku_add, sku_rm, sku_buy, sku_add_cat, sku_rm_cat,
           sku_buy_cat, kde_add, kde_rms, kde_buys, kde_searchs, kde_visits,
           query_search, url_visit, W_actions, b_actions, W_kde_add,
           b_kde_add, W_kde_rms, b_kde_rms, W_kde_buys, b_kde_buys,
           W_kde_searchs, b_kde_searchs, W_kde_visits, b_kde_visits,
           sku_table, W_sku_add, b_sku_add, W_sku_rm, b_sku_rm, W_sku_buy,
           b_sku_buy, W_query, b_query, cat_table, W_cat_add, b_cat_add,
           W_cat_rm, b_cat_rm, W_cat_buy, b_cat_buy, url_table, bn_gamma,
           bn_beta, W_full, b_full):
    idx_s = jnp.stack([sku_add, sku_rm, sku_buy, sku_add_cat, sku_rm_cat,
                       sku_buy_cat, url_visit]).astype(jnp.int32)
    idx_p = idx_s if LP == L else jnp.concatenate(
        [idx_s, jnp.zeros((NBAG, B, LP - L), jnp.int32)], axis=2)
    idx_r = idx_p.transpose(0, 2, 1).reshape(NBAG, LP, NW, CB)
    idx_r = idx_r.transpose(0, 2, 1, 3).reshape(NBAG, NW, LP * CB)

    bags = _sc_bag_sums(sku_table, cat_table, url_table, idx_r)

    kdes = jnp.stack([kde_add, kde_rms, kde_buys, kde_searchs, kde_visits])
    w_kde = jnp.stack([W_kde_add, W_kde_rms, W_kde_buys, W_kde_searchs,
                       W_kde_visits])
    b_kde = jnp.stack([b_kde_add, b_kde_rms, b_kde_buys, b_kde_searchs,
                       b_kde_visits])
    w_bag = jnp.stack([W_sku_add, W_sku_rm, W_sku_buy, W_cat_add, W_cat_rm,
                       W_cat_buy])
    b_bag = jnp.stack([b_sku_add, b_sku_rm, b_sku_buy, b_cat_add, b_cat_rm,
                       b_cat_buy])

    return _tc_encode(bags, idx_s, actions, kdes, query_search,
                      W_actions, b_actions, w_kde, b_kde, W_query, b_query,
                      w_bag, b_bag, bn_gamma, bn_beta,
                      W_full.astype(jnp.bfloat16), b_full)


# speedup vs baseline: 1.0207x; 1.0207x over previous
import functools

import jax
import jax.numpy as jnp
from jax import lax
from jax.experimental import pallas as pl
from jax.experimental.pallas import tpu as pltpu
from jax.experimental.pallas import tpu_sc as plsc

B, L, H = 4096, 50, 128
UEMB = 512
NBAG = 7
NCHUNK = 14

KR = 2
LP = 50
NC, NS = 2, 16
NW = NC * NS
CB = B // NW


def _load_row(src_ref, i):
    return tuple(src_ref[i, pl.ds(c * 16, 16)] for c in range(H // 16))


def _store_row(acc_ref, i, xs, first):
    for c in range(H // 16):
        sl = pl.ds(c * 16, 16)
        if first:
            acc_ref[i, sl] = xs[c]
        else:
            plsc.addupdate(acc_ref.at[i, sl], xs[c])


def _accum(acc_ref, src_ref, first=False):
    def load_rows(i):
        xs = _load_row(src_ref, i)
        for r in range(1, KR):
            ys = _load_row(src_ref, r * CB + i)
            xs = tuple(x + y for x, y in zip(xs, ys))
        return xs

    first_vals = load_rows(0)

    @plsc.parallel_loop(1, CB, carry=first_vals, unroll=4)
    def body(i, xs):
        _store_row(acc_ref, i - 1, xs, first)
        return load_rows(i)

    _store_row(acc_ref, CB - 1, body, first)


def _sc_body(sku_t, cat_t, url_t, idx_r, out, idx_v, buf_v, acc_v, cat_sh,
             sem0, sem1, sem2):
    cid = lax.axis_index("c")
    sid = lax.axis_index("s")
    wid = sid * NC + cid
    b0 = wid * CB

    @pl.when(sid == 0)
    def _load_cat():
        pltpu.sync_copy(cat_t, cat_sh)

    plsc.subcore_barrier()

    tables = (sku_t, sku_t, sku_t, cat_sh, cat_sh, cat_sh, url_t)
    nstep = LP // KR

    for t in range(NBAG):
        table = tables[t]
        pltpu.sync_copy(idx_r.at[t, wid], idx_v)

        def start(s_, buf, sem):
            pltpu.async_copy(
                table.at[idx_v.at[pl.ds(s_ * KR * CB, KR * CB)]],
                buf_v.at[buf], sem)

        def retire(buf, sem, first=False):
            pltpu.make_async_copy(table.at[idx_v.at[pl.ds(0, KR * CB)]],
                                  buf_v.at[buf], sem).wait()
            _accum(acc_v, buf_v.at[buf], first=first)

        sems = (sem0, sem1, sem2)
        start(0, 0, sem0)
        start(1, 1, sem1)
        start(2, 2, sem2)
        retire(0, sem0, first=True)

        def triple(j, carry):
            for m in range(3):
                s_ = 3 * j + 3 + m
                start(s_, m, sems[m])
                retire((m + 1) % 3, sems[(m + 1) % 3])
            return carry

        lax.fori_loop(0, (nstep - 4) // 3, triple, 0)
        start(nstep - 1, 0, sem0)
        retire(1, sem1)
        retire(2, sem2)
        retire(0, sem0)

        pltpu.sync_copy(acc_v, out.at[t, pl.ds(b0, CB), :])


@functools.partial(jax.jit, static_argnames=())
def _sc_bag_sums(sku_table, cat_table, url_table, idx_r):
    mesh = plsc.VectorSubcoreMesh(core_axis_name="c", subcore_axis_name="s")
    return pl.kernel(
        _sc_body,
        out_type=jax.ShapeDtypeStruct((NBAG, B, H), jnp.float32),
        mesh=mesh,
        scratch_types=[
            pltpu.VMEM((LP * CB,), jnp.int32),
            pltpu.VMEM((3, KR * CB, H), jnp.float32),
            pltpu.VMEM((CB, H), jnp.float32),
            pltpu.VMEM_SHARED((1001, H), jnp.float32),
            pltpu.SemaphoreType.DMA,
            pltpu.SemaphoreType.DMA,
            pltpu.SemaphoreType.DMA,
        ],
    )(sku_table, cat_table, url_table, idx_r)


def _mm(x, w):
    return lax.dot_general(x, w, (((1,), (1,)), ((), ())),
                           preferred_element_type=jnp.float32)


def _tc_body(bags, idxs, actions, kdes, query,
             w_act, b_act, w_kde, b_kde, w_query, b_query,
             w_bag, b_bag, gamma, beta, w_full, b_full, out):
    f32 = jnp.float32
    means = []
    for t in range(NBAG):
        cnt = jnp.sum((idxs[t] != 0).astype(f32), axis=1)
        inv = 1.0 / jnp.maximum(cnt, 1.0)
        means.append(bags[t] * inv[:, None])

    chunks = [
        _mm(actions[...], w_act[...]) + b_act[...][None, :],
        _mm(kdes[0], w_kde[0]) + b_kde[0][None, :],
        _mm(kdes[1], w_kde[1]) + b_kde[1][None, :],
        _mm(kdes[2], w_kde[2]) + b_kde[2][None, :],
        _mm(kdes[3], w_kde[3]) + b_kde[3][None, :],
        _mm(kdes[4], w_kde[4]) + b_kde[4][None, :],
        _mm(means[0], w_bag[0]) + b_bag[0][None, :],
        _mm(means[1], w_bag[1]) + b_bag[1][None, :],
        _mm(means[2], w_bag[2]) + b_bag[2][None, :],
        _mm(means[3], w_bag[3]) + b_bag[3][None, :],
        _mm(means[4], w_bag[4]) + b_bag[4][None, :],
        _mm(means[5], w_bag[5]) + b_bag[5][None, :],
        _mm(query[...], w_query[...]) + b_query[...][None, :],
        means[6],
    ]

    bn_s = 1.0 / jnp.sqrt(1.0 + 1e-5)
    acc = jnp.zeros(out.shape, f32) + b_full[...][None, :]
    for ci in range(NCHUNK):
        g = gamma[pl.ds(ci * H, H)] * bn_s
        be = beta[pl.ds(ci * H, H)]
        x = chunks[ci] * g[None, :] + be[None, :]
        acc = acc + _mm(x.astype(jnp.bfloat16), w_full[:, pl.ds(ci * H, H)])
    out[...] = acc


def _tc_encode(bags, idx_s, actions, kdes, query,
               w_act, b_act, w_kde, b_kde, w_query, b_query,
               w_bag, b_bag, gamma, beta, w_full, b_full):
    TB = 256
    grid = (B // TB,)
    batch2 = lambda i: (i, 0)
    batch3 = lambda i: (0, i, 0)
    whole = lambda i: (0,)
    return pl.pallas_call(
        _tc_body,
        grid=grid,
        in_specs=[
            pl.BlockSpec((NBAG, TB, H), batch3),
            pl.BlockSpec((NBAG, TB, L), batch3),
            pl.BlockSpec((TB, 5), batch2),
            pl.BlockSpec((5, TB, 11), batch3),
            pl.BlockSpec((TB, 64), batch2),
            pl.BlockSpec((H, 5), lambda i: (0, 0)),
            pl.BlockSpec((H,), whole),
            pl.BlockSpec((5, H, 11), lambda i: (0, 0, 0)),
            pl.BlockSpec((5, H), lambda i: (0, 0)),
            pl.BlockSpec((H, 64), lambda i: (0, 0)),
            pl.BlockSpec((H,), whole),
            pl.BlockSpec((6, H, H), lambda i: (0, 0, 0)),
            pl.BlockSpec((6, H), lambda i: (0, 0)),
            pl.BlockSpec((NCHUNK * H,), whole),
            pl.BlockSpec((NCHUNK * H,), whole),
            pl.BlockSpec((UEMB, NCHUNK * H), lambda i: (0, 0)),
            pl.BlockSpec((UEMB,), whole),
        ],
        out_specs=pl.BlockSpec((TB, UEMB), batch2),
        out_shape=jax.ShapeDtypeStruct((B, UEMB), jnp.float32),
    )(bags, idx_s, actions, kdes, query, w_act, b_act, w_kde, b_kde,
      w_query, b_query, w_bag, b_bag, gamma, beta, w_full, b_full)


def kernel(actions, sku_add, sku_rm, sku_buy, sku_add_cat, sku_rm_cat,
           sku_buy_cat, kde_add, kde_rms, kde_buys, kde_searchs, kde_visits,
           query_search, url_visit, W_actions, b_actions, W_kde_add,
           b_kde_add, W_kde_rms, b_kde_rms, W_kde_buys, b_kde_buys,
           W_kde_searchs, b_kde_searchs, W_kde_visits, b_kde_visits,
           sku_table, W_sku_add, b_sku_add, W_sku_rm, b_sku_rm, W_sku_buy,
           b_sku_buy, W_query, b_query, cat_table, W_cat_add, b_cat_add,
           W_cat_rm, b_cat_rm, W_cat_buy, b_cat_buy, url_table, bn_gamma,
           bn_beta, W_full, b_full):
    idx_s = jnp.stack([sku_add, sku_rm, sku_buy, sku_add_cat, sku_rm_cat,
                       sku_buy_cat, url_visit]).astype(jnp.int32)
    idx_p = idx_s if LP == L else jnp.concatenate(
        [idx_s, jnp.zeros((NBAG, B, LP - L), jnp.int32)], axis=2)
    idx_r = idx_p.transpose(0, 2, 1).reshape(NBAG, LP, NW, CB)
    idx_r = idx_r.transpose(0, 2, 1, 3).reshape(NBAG, NW, LP * CB)

    bags = _sc_bag_sums(sku_table, cat_table, url_table, idx_r)

    kdes = jnp.stack([kde_add, kde_rms, kde_buys, kde_searchs, kde_visits])
    w_kde = jnp.stack([W_kde_add, W_kde_rms, W_kde_buys, W_kde_searchs,
                       W_kde_visits])
    b_kde = jnp.stack([b_kde_add, b_kde_rms, b_kde_buys, b_kde_searchs,
                       b_kde_visits])
    w_bag = jnp.stack([W_sku_add, W_sku_rm, W_sku_buy, W_cat_add, W_cat_rm,
                       W_cat_buy])
    b_bag = jnp.stack([b_sku_add, b_sku_rm, b_sku_buy, b_cat_add, b_cat_rm,
                       b_cat_buy])

    return _tc_encode(bags, idx_s, actions, kdes, query_search,
                      W_actions, b_actions, w_kde, b_kde, W_query, b_query,
                      w_bag, b_bag, bn_gamma, bn_beta,
                      W_full.astype(jnp.bfloat16), b_full)
